# baseline (speedup 1.0000x reference)
import functools

import jax
import jax.numpy as jnp
from jax.experimental import pallas as pl
from jax.experimental.pallas import tpu as pltpu

_S = 32.0
_M = 0.5
_LOG2E = 1.4426950408889634
_LN2 = 0.6931471805599453


def _loss_kernel(lab_ref, x_ref, out_ref, *, b_total, g, s_scale, margin):
    r = pl.program_id(0)
    nr = pl.num_programs(0)
    c1 = s_scale * _LOG2E
    dm = s_scale * margin * _LOG2E

    z = x_ref[...] * c1
    lab = lab_ref[0, 0, :][:, None]
    cols = jax.lax.broadcasted_iota(jnp.int32, z.shape, 1)
    is_lab = cols == lab
    zl_sum = jnp.sum(jnp.where(is_lab, z, 0.0), axis=1)
    z = jnp.where(is_lab, z - dm, z)
    m_sub = jnp.max(z, axis=1)
    s_sub = jnp.sum(jnp.exp2(z - m_sub[:, None]), axis=1)

    mg = m_sub.reshape(-1, g)
    sg = s_sub.reshape(-1, g)
    tg = zl_sum.reshape(-1, g)
    m_row = jnp.max(mg, axis=1)
    s_row = jnp.sum(sg * jnp.exp2(mg - m_row[:, None]), axis=1)
    t_row = jnp.sum(tg, axis=1) - dm
    part = _LN2 * jnp.sum(m_row + jnp.log2(s_row) - t_row)

    prev = jnp.where(r == 0, 0.0, out_ref[0, 0])
    tot = prev + part
    out_ref[0, 0] = jnp.where(r == nr - 1, tot / b_total, tot)


def kernel(input, labels):
    b, c_total = input.shape
    w = max(d for d in range(1, 2049) if c_total % d == 0)
    g = c_total // w
    x2 = input.reshape(b * g, w)

    r_orig = 8
    while b % r_orig != 0 or (r_orig * g) % 8 != 0:
        r_orig *= 2
    r2 = r_orig * g
    nr = (b * g) // r2

    lab32 = labels.astype(jnp.int32)
    labadj = (jnp.repeat(lab32, g)
              - jnp.tile(jnp.arange(g, dtype=jnp.int32) * w, (b,)))
    labadj3 = labadj.reshape(nr, 1, r2)

    out = pl.pallas_call(
        functools.partial(_loss_kernel, b_total=b, g=g, s_scale=_S,
                          margin=_M),
        grid=(nr,),
        in_specs=[
            pl.BlockSpec((1, 1, r2), lambda r: (r, 0, 0)),
            pl.BlockSpec((r2, w), lambda r: (r, 0)),
        ],
        out_specs=pl.BlockSpec((1, 1), lambda r: (0, 0),
                               memory_space=pltpu.SMEM),
        out_shape=jax.ShapeDtypeStruct((1, 1), jnp.float32),
        compiler_params=pltpu.CompilerParams(
            dimension_semantics=("arbitrary",)),
    )(labadj3, x2)
    return out[0, 0]


# speedup vs baseline: 1.6531x; 1.6531x over previous
import functools

import jax
import jax.numpy as jnp
from jax.experimental import pallas as pl
from jax.experimental.pallas import tpu as pltpu

_S = 32.0
_M = 0.5
_LOG2E = 1.4426950408889634
_LN2 = 0.6931471805599453


def _loss_kernel(lab_ref, x_ref, out_ref, m_ref, s_ref, t_ref, *, cb,
                 c_total, b_total, s_scale, margin):
    c = pl.program_id(0)
    nc = pl.num_programs(0)
    c1 = s_scale * _LOG2E
    dm = s_scale * margin * _LOG2E

    @pl.when(c == 0)
    def _init():
        m_ref[...] = jnp.full_like(m_ref, -jnp.inf)
        s_ref[...] = jnp.zeros_like(s_ref)
        t_ref[...] = jnp.zeros_like(t_ref)

    lab = lab_ref[0, 0, :][:, None]

    def update(mask_pad):
        z = x_ref[...] * c1
        cols = c * cb + jax.lax.broadcasted_iota(jnp.int32, z.shape, 1)
        is_lab = cols == lab
        z = jnp.where(is_lab, z - dm, z)
        if mask_pad:
            z = jnp.where(cols < c_total, z, -jnp.inf)
        t_ref[...] += jnp.sum(jnp.where(is_lab, z, 0.0), axis=1,
                              keepdims=True)
        m_prev = m_ref[...]
        m_new = jnp.maximum(m_prev, jnp.max(z, axis=1, keepdims=True))
        s_ref[...] = s_ref[...] * jnp.exp2(m_prev - m_new) + jnp.sum(
            jnp.exp2(z - m_new), axis=1, keepdims=True)
        m_ref[...] = m_new

    @pl.when(c < nc - 1)
    def _main():
        update(False)

    @pl.when(c == nc - 1)
    def _tail():
        update(True)
        lse2 = m_ref[...] + jnp.log2(s_ref[...])
        out_ref[0, 0] = _LN2 * jnp.sum(lse2 - t_ref[...]) / b_total


def kernel(input, labels):
    b, c_total = input.shape
    cb = 2048
    nc = pl.cdiv(c_total, cb)

    lab3 = labels.astype(jnp.int32).reshape(1, 1, b)

    out = pl.pallas_call(
        functools.partial(_loss_kernel, cb=cb, c_total=c_total, b_total=b,
                          s_scale=_S, margin=_M),
        grid=(nc,),
        in_specs=[
            pl.BlockSpec((1, 1, b), lambda c: (0, 0, 0)),
            pl.BlockSpec((b, cb), lambda c: (0, c)),
        ],
        out_specs=pl.BlockSpec((1, 1), lambda c: (0, 0),
                               memory_space=pltpu.SMEM),
        out_shape=jax.ShapeDtypeStruct((1, 1), jnp.float32),
        scratch_shapes=[
            pltpu.VMEM((b, 1), jnp.float32),
            pltpu.VMEM((b, 1), jnp.float32),
            pltpu.VMEM((b, 1), jnp.float32),
        ],
        compiler_params=pltpu.CompilerParams(
            dimension_semantics=("arbitrary",)),
    )(lab3, input)
    return out[0, 0]


# speedup vs baseline: 1.7156x; 1.0378x over previous
import functools

import jax
import jax.numpy as jnp
from jax import lax
from jax.experimental import pallas as pl
from jax.experimental.pallas import tpu as pltpu
from jax.experimental.pallas import tpu_sc as plsc

_S = 32.0
_M = 0.5
_LOG2E = 1.4426950408889634
_LN2 = 0.6931471805599453


def _sc_gather_body(x_hbm, lab_hbm, out_hbm, lab_v, chunk_v, *, rpw):
    wid = lax.axis_index("s") * 2 + lax.axis_index("c")
    base = wid * rpw
    pltpu.sync_copy(lab_hbm.at[pl.ds(base, rpw)], lab_v)
    for h in range(rpw // 16):
        labs = lab_v[pl.ds(h * 16, 16)]
        for j in range(16):
            row = base + h * 16 + j
            lab_j = labs[j]
            start = (lab_j // 16) * 16
            pltpu.sync_copy(x_hbm.at[row, pl.ds(start, 16)], chunk_v)
            pltpu.sync_copy(chunk_v, out_hbm.at[row])


def _sc_gather(input, labels):
    b, _ = input.shape
    rpw = b // 32
    mesh = plsc.VectorSubcoreMesh(core_axis_name="c", subcore_axis_name="s")
    return pl.kernel(
        functools.partial(_sc_gather_body, rpw=rpw),
        out_type=jax.ShapeDtypeStruct((b, 16), jnp.float32),
        mesh=mesh,
        scratch_types=[
            pltpu.VMEM((rpw,), jnp.int32),
            pltpu.VMEM((16,), jnp.float32),
        ],
    )(input, labels)


def _loss_kernel(lab_ref, xl_ref, x_ref, out_ref, m_ref, s_ref, *, cb, c_total,
                 b_total, s_scale, margin):
    c = pl.program_id(0)
    nc = pl.num_programs(0)
    c1 = s_scale * _LOG2E
    dm = s_scale * margin * _LOG2E

    @pl.when(c == 0)
    def _init():
        m_ref[...] = jnp.full_like(m_ref, -jnp.inf)
        s_ref[...] = jnp.zeros_like(s_ref)

    def update(mask_pad):
        z = x_ref[...] * c1
        if mask_pad:
            cols = c * cb + jax.lax.broadcasted_iota(jnp.int32, z.shape, 1)
            z = jnp.where(cols < c_total, z, -jnp.inf)
        m_prev = m_ref[...]
        m_new = jnp.maximum(m_prev, jnp.max(z, axis=1, keepdims=True))
        s_ref[...] = s_ref[...] * jnp.exp2(m_prev - m_new) + jnp.sum(
            jnp.exp2(z - m_new), axis=1, keepdims=True)
        m_ref[...] = m_new

    @pl.when(c < nc - 1)
    def _main():
        update(False)

    @pl.when(c == nc - 1)
    def _tail():
        update(True)
        lab = lab_ref[0, 0, :][:, None]
        off = lab - (lab // 16) * 16
        sub = jax.lax.broadcasted_iota(jnp.int32, (xl_ref.shape[0], 16), 1)
        xl = jnp.sum(jnp.where(sub == off, xl_ref[...], 0.0), axis=1,
                     keepdims=True)
        zl = xl * c1
        m = m_ref[...]
        s = s_ref[...] - jnp.exp2(zl - m) + jnp.exp2(zl - dm - m)
        lse2 = m + jnp.log2(s)
        out_ref[0, 0] = _LN2 * jnp.sum(lse2 - (zl - dm)) / b_total


def kernel(input, labels):
    b, c_total = input.shape
    cb = 2048
    nc = pl.cdiv(c_total, cb)

    lab32 = labels.astype(jnp.int32)
    xlc = _sc_gather(input, lab32)
    lab3 = lab32.reshape(1, 1, b)

    out = pl.pallas_call(
        functools.partial(_loss_kernel, cb=cb, c_total=c_total, b_total=b,
                          s_scale=_S, margin=_M),
        grid=(nc,),
        in_specs=[
            pl.BlockSpec((1, 1, b), lambda c: (0, 0, 0)),
            pl.BlockSpec((b, 16), lambda c: (0, 0)),
            pl.BlockSpec((b, cb), lambda c: (0, c)),
        ],
        out_specs=pl.BlockSpec((1, 1), lambda c: (0, 0),
                               memory_space=pltpu.SMEM),
        out_shape=jax.ShapeDtypeStruct((1, 1), jnp.float32),
        scratch_shapes=[
            pltpu.VMEM((b, 1), jnp.float32),
            pltpu.VMEM((b, 1), jnp.float32),
        ],
        compiler_params=pltpu.CompilerParams(
            dimension_semantics=("arbitrary",)),
    )(lab3, xlc, input)
    return out[0, 0]
